# baseline (speedup 1.0000x reference)
import functools

import jax
import jax.numpy as jnp
from jax import lax
from jax.experimental import pallas as pl
from jax.experimental.pallas import tpu as pltpu
from jax.experimental.pallas import tpu_sc as plsc

N, L, HALF = 16, 2048, 64
A = 14
DF = 2 * HALF
B = N * L * A
R = 21

NW = 32
B_PER_W = B // NW
CHUNK = 64
NBUF = 8
NCH = B_PER_W // CHUNK
NITER = NCH // NBUF


def _build_table2_kernel(rt_ref, at_ref, out_ref):
    rt = rt_ref[...]
    at = at_ref[...]
    out_ref[...] = jnp.concatenate(
        [
            jnp.broadcast_to(rt[:, None, :], (R, A, HALF)),
            jnp.broadcast_to(at[None, :, :], (R, A, HALF)),
        ],
        axis=-1,
    )


def _make_gather_kernel():
    mesh = plsc.VectorSubcoreMesh(core_axis_name="c", subcore_axis_name="s")
    scratch = [pltpu.VMEM((NCH, CHUNK), jnp.int32)]
    scratch += [pltpu.VMEM((CHUNK, DF), jnp.float32) for _ in range(NBUF)]
    scratch += [pltpu.SemaphoreType.DMA for _ in range(2 * NBUF)]

    @functools.partial(
        pl.kernel,
        mesh=mesh,
        out_type=jax.ShapeDtypeStruct((N, L * A, DF), jnp.float32),
        scratch_types=scratch,
    )
    def gather_kernel(idx_hbm, table_hbm, out_hbm, idx_v, *rest):
        bufs = rest[:NBUF]
        gsems = rest[NBUF:2 * NBUF]
        ssems = rest[2 * NBUF:]
        wid = lax.axis_index("s") * 2 + lax.axis_index("c")
        n = wid // 2
        off = (wid % 2) * B_PER_W
        pltpu.sync_copy(idx_hbm.at[wid], idx_v)

        def start_gather(c, b):
            pltpu.async_copy(table_hbm.at[idx_v.at[c]], bufs[b], gsems[b])

        def wait_gather(c, b):
            pltpu.make_async_copy(
                table_hbm.at[idx_v.at[c]], bufs[b], gsems[b]).wait()

        def start_scatter(c, b):
            pltpu.async_copy(
                bufs[b], out_hbm.at[n, pl.ds(off + c * CHUNK, CHUNK)],
                ssems[b])

        def wait_scatter(c, b):
            pltpu.make_async_copy(
                bufs[b], out_hbm.at[n, pl.ds(off + c * CHUNK, CHUNK)],
                ssems[b]).wait()

        for b in range(NBUF):
            start_gather(b, b)

        def body(i, carry):
            c0 = i * NBUF
            for b in range(NBUF):
                wait_gather(c0 + b, b)
                start_scatter(c0 + b, b)
            for b in range(NBUF):
                wait_scatter(c0 + b, b)
                start_gather(c0 + NBUF + b, b)
            return carry

        lax.fori_loop(0, NITER - 1, body, 0)

        c0 = (NITER - 1) * NBUF
        for b in range(NBUF):
            wait_gather(c0 + b, b)
            start_scatter(c0 + b, b)
        for b in range(NBUF):
            wait_scatter(c0 + b, b)

    return gather_kernel


_GATHER = _make_gather_kernel()


def kernel(aa, pos14, atom_mask, residual_table, atom_table):
    table3 = pl.pallas_call(
        _build_table2_kernel,
        out_shape=jax.ShapeDtypeStruct((R, A, DF), jnp.float32),
    )(residual_table, atom_table)
    table2 = table3.reshape(R * A, DF)
    aa32 = aa.astype(jnp.int32)
    idx = (aa32[:, :, None] * A
           + jnp.arange(A, dtype=jnp.int32)).reshape(NW, NCH, CHUNK)
    feats = _GATHER(idx, table2)
    coors = pos14.reshape(N, L * A, 3)
    mask = atom_mask.reshape(N, L * A)
    return (feats, coors, mask)


# speedup vs baseline: 2.6378x; 2.6378x over previous
import functools

import jax
import jax.numpy as jnp
from jax import lax
from jax.experimental import pallas as pl
from jax.experimental.pallas import tpu as pltpu
from jax.experimental.pallas import tpu_sc as plsc

N, L, HALF = 16, 2048, 64
A = 14
DF = 2 * HALF
R = 21

NW = 32
RES_PER_W = N * L // NW
ROWS_PER_W = RES_PER_W * A


def _build_table2_kernel(rt_ref, at_ref, out_ref):
    rt = rt_ref[...]
    at = at_ref[...]
    out_ref[...] = jnp.concatenate(
        [
            jnp.broadcast_to(rt[:, None, :], (R, A, HALF)),
            jnp.broadcast_to(at[None, :, :], (R, A, HALF)),
        ],
        axis=-1,
    )


def _make_expand_kernel():
    mesh = plsc.VectorSubcoreMesh(core_axis_name="c", subcore_axis_name="s")

    @functools.partial(
        pl.kernel,
        mesh=mesh,
        out_type=jax.ShapeDtypeStruct((N, L * A, DF), jnp.float32),
        compiler_params=pltpu.CompilerParams(use_tc_tiling_on_sc=False),
        scratch_types=[
            pltpu.VMEM((RES_PER_W,), jnp.int32),
            pltpu.VMEM((R * A, DF), jnp.float32),
            pltpu.SemaphoreType.DMA,
        ],
    )
    def expand_kernel(aa_hbm, table_hbm, out_hbm, aa_v, table_v, osem):
        wid = lax.axis_index("s") * 2 + lax.axis_index("c")
        n = wid // 2
        off = (wid % 2) * ROWS_PER_W
        pltpu.sync_copy(table_hbm, table_v)
        pltpu.sync_copy(aa_hbm.at[n, pl.ds((wid % 2) * RES_PER_W, RES_PER_W)],
                        aa_v)

        def issue_group(g, carry):
            rows16 = aa_v[pl.ds(g * 16, 16)]
            base = off + g * 16 * A
            for k in range(16):
                row = rows16[k]
                pltpu.async_copy(
                    table_v.at[pl.ds(row * A, A)],
                    out_hbm.at[n, pl.ds(base + k * A, A)],
                    osem)
            return carry

        lax.fori_loop(0, RES_PER_W // 16, issue_group, 0)

        def drain(r, carry):
            pltpu.make_async_copy(
                table_v.at[pl.ds(0, A)],
                out_hbm.at[n, pl.ds(off + r * A, A)],
                osem).wait()
            return carry

        lax.fori_loop(0, RES_PER_W, drain, 0)

    return expand_kernel


_EXPAND = _make_expand_kernel()


def kernel(aa, pos14, atom_mask, residual_table, atom_table):
    table3 = pl.pallas_call(
        _build_table2_kernel,
        out_shape=jax.ShapeDtypeStruct((R, A, DF), jnp.float32),
    )(residual_table, atom_table)
    table2 = table3.reshape(R * A, DF)
    feats = _EXPAND(aa.astype(jnp.int32), table2)
    coors = pos14.reshape(N, L * A, 3)
    mask = atom_mask.reshape(N, L * A)
    return (feats, coors, mask)
